# speedup vs baseline: 2.1291x; 2.1291x over previous
import functools

import jax
import jax.numpy as jnp
from jax import lax
from jax.experimental import pallas as pl
from jax.experimental.pallas import tpu as pltpu
from jax.experimental.pallas import tpu_sc as plsc

N_TILE = 512
C_TILE = 2048
GATHER_CHUNK = 128


def _dist_argmax_body(x_ref, e_ref, dist_ref, ind_ref, max_scr, arg_scr):
    ci = pl.program_id(0)
    ni = pl.program_id(1)
    blk = lax.dot_general(
        x_ref[...], e_ref[...],
        (((1,), (1,)), ((), ())),
        preferred_element_type=jnp.float32,
    )
    dist_ref[...] = blk
    m = jnp.max(blk, axis=1, keepdims=True)
    iota = lax.broadcasted_iota(jnp.int32, blk.shape, 1)
    loc = jnp.min(jnp.where(blk == m, iota, blk.shape[1]), axis=1,
                  keepdims=True)
    gidx = loc + ci * C_TILE

    sl = pl.ds(ni * N_TILE, N_TILE)
    prev_m = max_scr[sl, :]
    prev_a = arg_scr[sl, :]
    upd = jnp.logical_or(ci == 0, m > prev_m)
    new_m = jnp.where(upd, m, prev_m)
    new_a = jnp.where(upd, gidx, prev_a)
    max_scr[sl, :] = new_m
    arg_scr[sl, :] = new_a
    ind_ref[...] = new_a


def _dist_argmax(xf, e, interpret=False):
    n, d = xf.shape
    c, _ = e.shape
    grid = (c // C_TILE, n // N_TILE)
    return pl.pallas_call(
        _dist_argmax_body,
        grid=grid,
        in_specs=[
            pl.BlockSpec((N_TILE, d), lambda ci, ni: (ni, 0)),
            pl.BlockSpec((C_TILE, d), lambda ci, ni: (ci, 0)),
        ],
        out_specs=[
            pl.BlockSpec((N_TILE, C_TILE), lambda ci, ni: (ni, ci)),
            pl.BlockSpec((N_TILE, 1), lambda ci, ni: (ni, 0)),
        ],
        out_shape=[
            jax.ShapeDtypeStruct((n, c), jnp.float32),
            jax.ShapeDtypeStruct((n, 1), jnp.int32),
        ],
        scratch_shapes=[
            pltpu.VMEM((n, 1), jnp.float32),
            pltpu.VMEM((n, 1), jnp.int32),
        ],
        interpret=interpret,
    )(xf, e)


def _sc_gather(table, idx):
    info = plsc.get_sparse_core_info()
    nw = info.num_cores * info.num_subcores
    b = idx.shape[0]
    d = table.shape[1]
    b_per_w = b // nw
    n_chunks = b_per_w // GATHER_CHUNK
    mesh = plsc.VectorSubcoreMesh(core_axis_name="c", subcore_axis_name="s")

    @functools.partial(
        pl.kernel,
        mesh=mesh,
        out_type=jax.ShapeDtypeStruct((b, d), jnp.float32),
        scratch_types=[
            pltpu.VMEM((GATHER_CHUNK,), jnp.int32),
            pltpu.VMEM((GATHER_CHUNK, d), jnp.float32),
            pltpu.SemaphoreType.DMA,
        ],
    )
    def k(table_hbm, idx_hbm, out_hbm, idx_v, rows_v, sem):
        wid = lax.axis_index("s") * info.num_cores + lax.axis_index("c")
        base = wid * b_per_w
        for i in range(n_chunks):
            off = base + i * GATHER_CHUNK
            pltpu.sync_copy(idx_hbm.at[pl.ds(off, GATHER_CHUNK)], idx_v)
            pltpu.async_copy(table_hbm.at[idx_v], rows_v, sem).wait()
            pltpu.sync_copy(rows_v, out_hbm.at[pl.ds(off, GATHER_CHUNK)])

    return k(table, idx)


def kernel(x, embed):
    b, n, d = x.shape
    h, c, _ = embed.shape
    xf = x.reshape(b * n, d)
    e = embed.reshape(c, d)
    dist2d, ind2d = _dist_argmax(xf, e)
    ind = ind2d.reshape(b * n)
    quantize = _sc_gather(e, ind).reshape(b, n, d)
    embed_ind = ind.reshape(b, n)
    dist_out = dist2d.reshape(h, b, n, c)
    return quantize, embed_ind, dist_out
